# baseline (speedup 1.0000x reference)
import functools

import jax
import jax.numpy as jnp
from jax.experimental import pallas as pl
from jax.experimental.pallas import tpu as pltpu

E = 8
TOP_K = 2
D = 2048
D_FF = 768
T = 2048
BLK_T = 256
NT = T // BLK_T


def _moe_dense_kernel(x_ref, gw_ref, wg_ref, wu_ref, wd_ref, out_ref):
    e = pl.program_id(1)
    x = x_ref[...]
    gw = gw_ref[...]
    logits = jax.lax.dot_general(x, gw, (((1,), (1,)), ((), ())),
                                 preferred_element_type=jnp.float32)
    iota = jax.lax.broadcasted_iota(jnp.int32, logits.shape, 1)
    m1 = jnp.max(logits, axis=-1, keepdims=True)
    a1 = jnp.min(jnp.where(logits == m1, iota, E), axis=-1, keepdims=True)
    l2 = jnp.where(iota == a1, -jnp.inf, logits)
    m2 = jnp.max(l2, axis=-1, keepdims=True)
    a2 = jnp.min(jnp.where(l2 == m2, iota, E), axis=-1, keepdims=True)
    w1 = 1.0 / (1.0 + jnp.exp(m2 - m1))
    w2 = 1.0 - w1
    coef = jnp.where(a1 == e, w1, 0.0) + jnp.where(a2 == e, w2, 0.0)

    wg = wg_ref[0]
    wu = wu_ref[0]
    wd = wd_ref[0]
    g = jax.lax.dot_general(x, wg, (((1,), (1,)), ((), ())),
                            preferred_element_type=jnp.float32)
    u = jax.lax.dot_general(x, wu, (((1,), (1,)), ((), ())),
                            preferred_element_type=jnp.float32)
    h = (g / (1.0 + jnp.exp(-g))) * u
    y = jax.lax.dot_general(h, wd, (((1,), (1,)), ((), ())),
                            preferred_element_type=jnp.float32)
    contrib = coef * y

    @pl.when(e == 0)
    def _():
        out_ref[...] = contrib

    @pl.when(e > 0)
    def _():
        out_ref[...] += contrib


def _moe_dense(x, gate_w, gate_proj_w, up_proj_w, down_proj_w):
    return pl.pallas_call(
        _moe_dense_kernel,
        grid=(NT, E),
        in_specs=[
            pl.BlockSpec((BLK_T, D), lambda t, e: (t, 0)),
            pl.BlockSpec((E, D), lambda t, e: (0, 0)),
            pl.BlockSpec((1, D_FF, D), lambda t, e: (e, 0, 0)),
            pl.BlockSpec((1, D_FF, D), lambda t, e: (e, 0, 0)),
            pl.BlockSpec((1, D, D_FF), lambda t, e: (e, 0, 0)),
        ],
        out_specs=pl.BlockSpec((BLK_T, D), lambda t, e: (t, 0)),
        out_shape=jax.ShapeDtypeStruct((T, D), jnp.float32),
        compiler_params=pltpu.CompilerParams(
            dimension_semantics=("parallel", "arbitrary"),
        ),
    )(x, gate_w, gate_proj_w, up_proj_w, down_proj_w)


def kernel(hidden_states, gate_w, gate_proj_w, up_proj_w, down_proj_w):
    b, s, d = hidden_states.shape
    x = hidden_states.reshape(-1, d)
    out = _moe_dense(x, gate_w, gate_proj_w, up_proj_w, down_proj_w)
    return out.reshape(b, s, d)


# speedup vs baseline: 1.1220x; 1.1220x over previous
import functools

import jax
import jax.numpy as jnp
from jax import lax
from jax.experimental import pallas as pl
from jax.experimental.pallas import tpu as pltpu
from jax.experimental.pallas import tpu_sc as plsc

E = 8
D = 2048
DFF = 768
T = 2048
NA = 2 * T
BLK = 128
NB = NA // BLK + E
P = NB * BLK
NBE = 48
NW = 32
ROWS_W = P // NW
GC = 16
TOK_W = T // NW
CC = 8


def _router_body(x_ref, gw_ref, e1_ref, e2_ref, w1_ref, w2_ref):
    x = x_ref[...]
    gw = gw_ref[...]
    logits = lax.dot_general(x, gw, (((1,), (1,)), ((), ())),
                             preferred_element_type=jnp.float32)
    iota = lax.broadcasted_iota(jnp.int32, logits.shape, 1)
    m1 = jnp.max(logits, axis=-1, keepdims=True)
    a1 = jnp.min(jnp.where(logits == m1, iota, E), axis=-1, keepdims=True)
    lm = jnp.where(iota == a1, -jnp.inf, logits)
    m2 = jnp.max(lm, axis=-1, keepdims=True)
    a2 = jnp.min(jnp.where(lm == m2, iota, E), axis=-1, keepdims=True)
    w1 = 1.0 / (1.0 + jnp.exp(m2 - m1))
    e1_ref[...] = a1[:, 0]
    e2_ref[...] = a2[:, 0]
    w1_ref[...] = w1[:, 0]
    w2_ref[...] = 1.0 - w1[:, 0]


def _router(x, gate_w):
    return pl.pallas_call(
        _router_body,
        out_shape=[
            jax.ShapeDtypeStruct((T,), jnp.int32),
            jax.ShapeDtypeStruct((T,), jnp.int32),
            jax.ShapeDtypeStruct((T,), jnp.float32),
            jax.ShapeDtypeStruct((T,), jnp.float32),
        ],
    )(x, gate_w)


def _route_body(e1_hbm, e2_hbm,
                tok_hbm, pos_hbm, be_hbm,
                eid_v, tokbuf, posbuf, becalc, zbuf):
    c = lax.axis_index("c")
    s = lax.axis_index("s")
    is_w = jnp.logical_and(c == 0, s < E)
    is_0 = jnp.logical_and(c == 0, s == 0)
    iota16 = lax.iota(jnp.int32, 16)
    zeros16 = jnp.zeros((16,), jnp.int32)

    @pl.when(is_w)
    def _():
        def z1(i, _):
            tokbuf[pl.ds(i * 16, 16)] = zeros16
            posbuf[pl.ds(i * 16, 16)] = zeros16
            posbuf[pl.ds((i + 128) * 16, 16)] = zeros16
            return 0
        lax.fori_loop(0, 128, z1, 0)

        pltpu.sync_copy(e1_hbm, eid_v.at[pl.ds(0, T)])
        pltpu.sync_copy(e2_hbm, eid_v.at[pl.ds(T, T)])

        def cbody(i, accs):
            v = eid_v[pl.ds(i * 16, 16)]
            return tuple(a + jnp.where(v == e, 1, 0)
                         for e, a in enumerate(accs))
        accs = lax.fori_loop(0, NA // 16, cbody, (zeros16,) * E)
        cl = zeros16
        for e in range(E):
            cl = jnp.where(iota16 == e, jnp.sum(accs[e]), cl)
        pc = ((cl + (BLK - 1)) >> 7) << 7
        cum = plsc.cumsum(pc)
        starts = cum - pc
        start_e = jnp.sum(jnp.where(iota16 == s, starts, 0))
        nch = jnp.sum(jnp.where(iota16 == s, pc, 0)) >> 7

        def sbody(i, run):
            v = eid_v[pl.ds(i * 16, 16)]
            m = v == s
            mi = jnp.where(m, 1, 0)
            incl = plsc.cumsum(mi)
            r_local = run + incl - 1
            a = i * 16 + iota16
            t = jnp.bitwise_and(a, T - 1)
            plsc.store_scatter(tokbuf, [r_local], t, mask=m)
            posbuf[pl.ds(i * 16, 16)] = jnp.where(m, start_e + r_local, 0)
            return run + jnp.sum(mi)
        lax.fori_loop(0, NA // 16, sbody, 0)

        def ch(j, _):
            off = pl.multiple_of(start_e + j * BLK, BLK)
            pltpu.sync_copy(tokbuf.at[pl.ds(j * BLK, BLK)],
                            tok_hbm.at[pl.ds(off, BLK)])
            return 0
        lax.fori_loop(0, nch, ch, 0)

        pltpu.sync_copy(posbuf, pos_hbm.at[s])

        @pl.when(s == 0)
        def _():
            eb = cum >> 7
            nbu = jnp.sum(jnp.where(iota16 == (E - 1), eb, 0))
            for v in range(NBE // 16):
                bv = v * 16 + iota16
                acc = zeros16
                for r in range(E):
                    end_r = jnp.sum(jnp.where(iota16 == r, eb, 0))
                    acc = acc + jnp.where(bv >= end_r, 1, 0)
                bev = jnp.minimum(acc, E - 1)
                bev = jnp.where(bv == NB, nbu, bev)
                becalc[pl.ds(v * 16, 16)] = bev
            pltpu.sync_copy(becalc, be_hbm)

            def zt(i, _):
                zbuf[pl.ds(i * 16, 16)] = zeros16
                return 0
            lax.fori_loop(0, BLK // 16, zt, 0)
            nbu = jnp.sum(jnp.where(iota16 == (E - 1), eb, 0))

            def ztail(j, _):
                off = pl.multiple_of(j * BLK, BLK)
                pltpu.sync_copy(zbuf, tok_hbm.at[pl.ds(off, BLK)])
                return 0
            lax.fori_loop(nbu, NB, ztail, 0)


def _route(e1, e2):
    mesh = plsc.VectorSubcoreMesh(core_axis_name="c", subcore_axis_name="s")
    f = pl.kernel(
        _route_body,
        out_type=[
            jax.ShapeDtypeStruct((P,), jnp.int32),
            jax.ShapeDtypeStruct((E, NA), jnp.int32),
            jax.ShapeDtypeStruct((NBE,), jnp.int32),
        ],
        mesh=mesh,
        compiler_params=pltpu.CompilerParams(needs_layout_passes=False),
        scratch_types=[
            pltpu.VMEM((NA,), jnp.int32),
            pltpu.VMEM((T,), jnp.int32),
            pltpu.VMEM((NA,), jnp.int32),
            pltpu.VMEM((NBE,), jnp.int32),
            pltpu.VMEM((BLK,), jnp.int32),
        ],
    )
    return f(e1, e2)


NBUF = 3
NB1 = 12
S1 = NB1 * BLK
S2 = P - S1


def _make_gather_body(r0, gw):
    def _gather_body(x_hbm, tok_hbm, xs_hbm, idx_v, rows, gsem, wsem):
        c = lax.axis_index("c")
        s = lax.axis_index("s")
        wid = c * 16 + s
        base = wid * gw
        pltpu.sync_copy(tok_hbm.at[pl.ds(r0 + base, gw)], idx_v)
        nch = gw // GC
        gcp = [None] * NBUF
        wcp = [None] * NBUF
        for j in range(min(NBUF - 1, nch)):
            gcp[j] = pltpu.async_copy(x_hbm.at[idx_v.at[pl.ds(j * GC, GC)]],
                                      rows.at[j], gsem[j])
        for j in range(nch):
            p = j % NBUF
            if j + NBUF - 1 < nch:
                q = (j + NBUF - 1) % NBUF
                if wcp[q] is not None:
                    wcp[q].wait()
                nxt = idx_v.at[pl.ds((j + NBUF - 1) * GC, GC)]
                gcp[q] = pltpu.async_copy(x_hbm.at[nxt], rows.at[q], gsem[q])
            gcp[p].wait()
            wcp[p] = pltpu.async_copy(rows.at[p],
                                      xs_hbm.at[pl.ds(base + j * GC, GC)],
                                      wsem[p])
        for j in range(max(0, nch - NBUF), nch):
            wcp[j % NBUF].wait()
    return _gather_body


def _gather_part(x, tok, r0, ph):
    gw = ph // NW
    mesh = plsc.VectorSubcoreMesh(core_axis_name="c", subcore_axis_name="s")
    f = pl.kernel(
        _make_gather_body(r0, gw),
        out_type=jax.ShapeDtypeStruct((ph, D), jnp.float32),
        mesh=mesh,
        compiler_params=pltpu.CompilerParams(needs_layout_passes=False),
        scratch_types=[
            pltpu.VMEM((gw,), jnp.int32),
            pltpu.VMEM((NBUF, GC, D), jnp.float32),
            [pltpu.SemaphoreType.DMA] * NBUF,
            [pltpu.SemaphoreType.DMA] * NBUF,
        ],
        name=f"gather_part{r0}",
    )
    return f(x, tok)


def _ffn_compute(xs, wg, wu, wd):
    g = lax.dot_general(xs, wg, (((1,), (1,)), ((), ())),
                        preferred_element_type=jnp.float32)
    u = lax.dot_general(xs, wu, (((1,), (1,)), ((), ())),
                        preferred_element_type=jnp.float32)
    h = (g / (1.0 + jnp.exp(-g))) * u
    return lax.dot_general(h, wd, (((1,), (1,)), ((), ())),
                           preferred_element_type=jnp.float32)


def _make_ffn_body_a(boff):
    def body(be_ref, xs_ref, wg_ref, wu_ref, wd_ref, ys_ref):
        @pl.when(pl.program_id(0) + boff < be_ref[NB])
        def _():
            ys_ref[...] = _ffn_compute(xs_ref[...], wg_ref[0], wu_ref[0],
                                       wd_ref[0])
    return body


def _make_ffn_body_b(boff):
    def body(be_ref, xs_ref, wg_ref, wu_ref, wd_ref, prev_ref, ys_ref):
        @pl.when(pl.program_id(0) + boff < be_ref[NB])
        def _():
            ys_ref[...] = _ffn_compute(xs_ref[...], wg_ref[0], wu_ref[0],
                                       wd_ref[0])
    return body


def _ffn_part(be, xs, gate_proj_w, up_proj_w, down_proj_w, boff, nbp,
              ys_prev):
    in_specs = [
        pl.BlockSpec((BLK, D), lambda b, be_ref: (b, 0)),
        pl.BlockSpec((1, DFF, D), lambda b, be_ref: (be_ref[b + boff], 0, 0)),
        pl.BlockSpec((1, DFF, D), lambda b, be_ref: (be_ref[b + boff], 0, 0)),
        pl.BlockSpec((1, D, DFF), lambda b, be_ref: (be_ref[b + boff], 0, 0)),
    ]
    args = [be, xs, gate_proj_w, up_proj_w, down_proj_w]
    aliases = {}
    body = _make_ffn_body_a(boff)
    if ys_prev is not None:
        in_specs.append(pl.BlockSpec(memory_space=pl.ANY))
        args.append(ys_prev)
        aliases = {5: 0}
        body = _make_ffn_body_b(boff)
    grid_spec = pltpu.PrefetchScalarGridSpec(
        num_scalar_prefetch=1,
        grid=(nbp,),
        in_specs=in_specs,
        out_specs=pl.BlockSpec((BLK, D), lambda b, be_ref: (b + boff, 0)),
    )
    return pl.pallas_call(
        body,
        grid_spec=grid_spec,
        out_shape=jax.ShapeDtypeStruct((P, D), jnp.float32),
        input_output_aliases=aliases,
        compiler_params=pltpu.CompilerParams(
            dimension_semantics=("arbitrary",),
        ),
    )(*args)


def _combine_body(ys_hbm, pos_hbm, w1_hbm, w2_hbm, out_hbm,
                  p1_v, p2_v, pv_t, w1_v, w2_v, rows_a, rows_b,
                  sem_a, sem_b, wsem):
    c = lax.axis_index("c")
    s = lax.axis_index("s")
    wid = c * 16 + s
    base = wid * TOK_W
    zeros16 = jnp.zeros((16,), jnp.int32)
    for k in range(TOK_W // 16):
        p1_v[pl.ds(k * 16, 16)] = zeros16
        p2_v[pl.ds(k * 16, 16)] = zeros16
    for w in range(E):
        pltpu.sync_copy(pos_hbm.at[w, pl.ds(base, TOK_W)], pv_t)
        for k in range(TOK_W // 16):
            p1_v[pl.ds(k * 16, 16)] += pv_t[pl.ds(k * 16, 16)]
        pltpu.sync_copy(pos_hbm.at[w, pl.ds(T + base, TOK_W)], pv_t)
        for k in range(TOK_W // 16):
            p2_v[pl.ds(k * 16, 16)] += pv_t[pl.ds(k * 16, 16)]
    pltpu.sync_copy(w1_hbm.at[pl.ds(base, TOK_W)], w1_v)
    pltpu.sync_copy(w2_hbm.at[pl.ds(base, TOK_W)], w2_v)
    wv1 = [w1_v[pl.ds(k * 16, 16)] for k in range(TOK_W // 16)]
    wv2 = [w2_v[pl.ds(k * 16, 16)] for k in range(TOK_W // 16)]

    nch = TOK_W // CC
    ga = [None, None]
    gb = [None, None]
    wb = [None, None]
    ga[0] = pltpu.async_copy(ys_hbm.at[p1_v.at[pl.ds(0, CC)]],
                             rows_a.at[0], sem_a[0])
    gb[0] = pltpu.async_copy(ys_hbm.at[p2_v.at[pl.ds(0, CC)]],
                             rows_b.at[0], sem_b[0])
    for j in range(nch):
        p = j % 2
        q = 1 - p
        if j + 1 < nch:
            if wb[q] is not None:
                wb[q].wait()
            ga[q] = pltpu.async_copy(
                ys_hbm.at[p1_v.at[pl.ds((j + 1) * CC, CC)]],
                rows_a.at[q], sem_a[q])
            gb[q] = pltpu.async_copy(
                ys_hbm.at[p2_v.at[pl.ds((j + 1) * CC, CC)]],
                rows_b.at[q], sem_b[q])
        ga[p].wait()
        gb[p].wait()
        for i in range(CC):
            t_i = j * CC + i
            w1s = wv1[t_i // 16][t_i % 16]
            w2s = wv2[t_i // 16][t_i % 16]

            def inner(m, _):
                for u in range(8):
                    a = rows_a[p, i, pl.ds(m * 128 + u * 16, 16)]
                    b = rows_b[p, i, pl.ds(m * 128 + u * 16, 16)]
                    rows_a[p, i, pl.ds(m * 128 + u * 16, 16)] = (
                        a * w1s + b * w2s)
                return 0
            lax.fori_loop(0, D // 128, inner, 0)
        wb[p] = pltpu.async_copy(rows_a.at[p],
                                 out_hbm.at[pl.ds(base + j * CC, CC)],
                                 wsem[p])
    wb[0].wait()
    wb[1].wait()


def _combine(ys, pos, w1, w2):
    mesh = plsc.VectorSubcoreMesh(core_axis_name="c", subcore_axis_name="s")
    f = pl.kernel(
        _combine_body,
        out_type=jax.ShapeDtypeStruct((T, D), jnp.float32),
        mesh=mesh,
        compiler_params=pltpu.CompilerParams(needs_layout_passes=False),
        scratch_types=[
            pltpu.VMEM((TOK_W,), jnp.int32),
            pltpu.VMEM((TOK_W,), jnp.int32),
            pltpu.VMEM((TOK_W,), jnp.int32),
            pltpu.VMEM((TOK_W,), jnp.float32),
            pltpu.VMEM((TOK_W,), jnp.float32),
            pltpu.VMEM((2, CC, D), jnp.float32),
            pltpu.VMEM((2, CC, D), jnp.float32),
            [pltpu.SemaphoreType.DMA, pltpu.SemaphoreType.DMA],
            [pltpu.SemaphoreType.DMA, pltpu.SemaphoreType.DMA],
            [pltpu.SemaphoreType.DMA, pltpu.SemaphoreType.DMA],
        ],
    )
    return f(ys, pos, w1, w2)


def _route_emul(e1, e2, w1, w2):
    eid = jnp.concatenate([e1, e2])
    wv = jnp.concatenate([w1, w2])
    cnt = jnp.zeros((E,), jnp.int32).at[eid].add(1)
    pcnt = ((cnt + BLK - 1) // BLK) * BLK
    cum = jnp.cumsum(pcnt)
    starts = cum - pcnt
    oh = (eid[:, None] == jnp.arange(E)[None, :]).astype(jnp.int32)
    rank = jnp.take_along_axis(jnp.cumsum(oh, axis=0) - 1, eid[:, None],
                               axis=1)[:, 0]
    pos = starts[eid] + rank
    a = jnp.arange(NA, dtype=jnp.int32)
    tok = jnp.zeros((P,), jnp.int32).at[pos].set(a & (T - 1))
    wsrt = jnp.zeros((P,), jnp.float32).at[pos].set(wv)
    ends = cum // BLK
    b = jnp.arange(NBE, dtype=jnp.int32)
    be = jnp.minimum(jnp.sum(b[:, None] >= ends[None, :], axis=1), E - 1)
    return tok, wsrt, pos.reshape(NA // 16, 16), be.astype(jnp.int32)


def _gather_emul(x, tok):
    return x[tok]


def _combine_emul(ys, pos, w1, w2):
    return w1[:, None] * ys[pos[:T]] + w2[:, None] * ys[pos[T:]]


def kernel(hidden_states, gate_w, gate_proj_w, up_proj_w, down_proj_w):
    b, s, d = hidden_states.shape
    x = hidden_states.reshape(-1, d)
    e1, e2, w1, w2 = _router(x, gate_w)
    tok, pos_part, be = _route(e1, e2)
    xsa = _gather_part(x, tok, 0, S1)
    xsb = _gather_part(x, tok, S1, S2)
    ysa = _ffn_part(be, xsa, gate_proj_w, up_proj_w, down_proj_w, 0, NB1,
                    None)
    ys = _ffn_part(be, xsb, gate_proj_w, up_proj_w, down_proj_w, NB1,
                   NB - NB1, ysa)
    out = _combine(ys, pos_part, w1, w2)
    return out.reshape(b, s, d)


# speedup vs baseline: 1.1390x; 1.0152x over previous
import functools

import jax
import jax.numpy as jnp
from jax import lax
from jax.experimental import pallas as pl
from jax.experimental.pallas import tpu as pltpu
from jax.experimental.pallas import tpu_sc as plsc

E = 8
D = 2048
DFF = 768
T = 2048
NA = 2 * T
BLK = 128
NB = NA // BLK + E
P = NB * BLK
NBE = 48
NW = 32
ROWS_W = P // NW
GC = 16
TOK_W = T // NW
CC = 8


def _router_body(x_ref, gw_ref, e1_ref, e2_ref, w1_ref, w2_ref):
    x = x_ref[...]
    gw = gw_ref[...]
    logits = lax.dot_general(x, gw, (((1,), (1,)), ((), ())),
                             preferred_element_type=jnp.float32)
    iota = lax.broadcasted_iota(jnp.int32, logits.shape, 1)
    m1 = jnp.max(logits, axis=-1, keepdims=True)
    a1 = jnp.min(jnp.where(logits == m1, iota, E), axis=-1, keepdims=True)
    lm = jnp.where(iota == a1, -jnp.inf, logits)
    m2 = jnp.max(lm, axis=-1, keepdims=True)
    a2 = jnp.min(jnp.where(lm == m2, iota, E), axis=-1, keepdims=True)
    w1 = 1.0 / (1.0 + jnp.exp(m2 - m1))
    e1_ref[...] = a1[:, 0]
    e2_ref[...] = a2[:, 0]
    w1_ref[...] = w1[:, 0]
    w2_ref[...] = 1.0 - w1[:, 0]


BT = 256


def _router(x, gate_w):
    return pl.pallas_call(
        _router_body,
        grid=(T // BT,),
        in_specs=[
            pl.BlockSpec((BT, D), lambda t: (t, 0)),
            pl.BlockSpec((E, D), lambda t: (0, 0)),
        ],
        out_specs=[
            pl.BlockSpec((BT,), lambda t: (t,)),
            pl.BlockSpec((BT,), lambda t: (t,)),
            pl.BlockSpec((BT,), lambda t: (t,)),
            pl.BlockSpec((BT,), lambda t: (t,)),
        ],
        out_shape=[
            jax.ShapeDtypeStruct((T,), jnp.int32),
            jax.ShapeDtypeStruct((T,), jnp.int32),
            jax.ShapeDtypeStruct((T,), jnp.float32),
            jax.ShapeDtypeStruct((T,), jnp.float32),
        ],
        compiler_params=pltpu.CompilerParams(
            dimension_semantics=("parallel",),
        ),
    )(x, gate_w)


def _route_body(e1_hbm, e2_hbm,
                tok_hbm, pos_hbm, be_hbm,
                eid_v, tokbuf, posbuf, becalc, zbuf):
    c = lax.axis_index("c")
    s = lax.axis_index("s")
    is_w = jnp.logical_and(c == 0, s < E)
    is_0 = jnp.logical_and(c == 0, s == 0)
    iota16 = lax.iota(jnp.int32, 16)
    zeros16 = jnp.zeros((16,), jnp.int32)

    @pl.when(is_w)
    def _():
        def z1(i, _):
            tokbuf[pl.ds(i * 16, 16)] = zeros16
            posbuf[pl.ds(i * 16, 16)] = zeros16
            posbuf[pl.ds((i + 128) * 16, 16)] = zeros16
            return 0
        lax.fori_loop(0, 128, z1, 0)

        pltpu.sync_copy(e1_hbm, eid_v.at[pl.ds(0, T)])
        pltpu.sync_copy(e2_hbm, eid_v.at[pl.ds(T, T)])

        def cbody(i, accs):
            v = eid_v[pl.ds(i * 16, 16)]
            return tuple(a + jnp.where(v == e, 1, 0)
                         for e, a in enumerate(accs))
        accs = lax.fori_loop(0, NA // 16, cbody, (zeros16,) * E)
        cl = zeros16
        for e in range(E):
            cl = jnp.where(iota16 == e, jnp.sum(accs[e]), cl)
        pc = ((cl + (BLK - 1)) >> 7) << 7
        cum = plsc.cumsum(pc)
        starts = cum - pc
        start_e = jnp.sum(jnp.where(iota16 == s, starts, 0))
        nch = jnp.sum(jnp.where(iota16 == s, pc, 0)) >> 7

        def sbody(i, run):
            v = eid_v[pl.ds(i * 16, 16)]
            m = v == s
            mi = jnp.where(m, 1, 0)
            incl = plsc.cumsum(mi)
            r_local = run + incl - 1
            a = i * 16 + iota16
            t = jnp.bitwise_and(a, T - 1)
            plsc.store_scatter(tokbuf, [r_local], t, mask=m)
            posbuf[pl.ds(i * 16, 16)] = jnp.where(m, start_e + r_local, 0)
            return run + jnp.sum(mi)
        lax.fori_loop(0, NA // 16, sbody, 0)

        def ch(j, _):
            off = pl.multiple_of(start_e + j * BLK, BLK)
            pltpu.sync_copy(tokbuf.at[pl.ds(j * BLK, BLK)],
                            tok_hbm.at[pl.ds(off, BLK)])
            return 0
        lax.fori_loop(0, nch, ch, 0)

        pltpu.sync_copy(posbuf, pos_hbm.at[s])

        @pl.when(s == 0)
        def _():
            eb = cum >> 7
            nbu = jnp.sum(jnp.where(iota16 == (E - 1), eb, 0))
            for v in range(NBE // 16):
                bv = v * 16 + iota16
                acc = zeros16
                for r in range(E):
                    end_r = jnp.sum(jnp.where(iota16 == r, eb, 0))
                    acc = acc + jnp.where(bv >= end_r, 1, 0)
                bev = jnp.minimum(acc, E - 1)
                bev = jnp.where(bv == NB, nbu, bev)
                becalc[pl.ds(v * 16, 16)] = bev
            pltpu.sync_copy(becalc, be_hbm)

            def zt(i, _):
                zbuf[pl.ds(i * 16, 16)] = zeros16
                return 0
            lax.fori_loop(0, BLK // 16, zt, 0)
            nbu = jnp.sum(jnp.where(iota16 == (E - 1), eb, 0))

            def ztail(j, _):
                off = pl.multiple_of(j * BLK, BLK)
                pltpu.sync_copy(zbuf, tok_hbm.at[pl.ds(off, BLK)])
                return 0
            lax.fori_loop(nbu, NB, ztail, 0)


def _route(e1, e2):
    mesh = plsc.VectorSubcoreMesh(core_axis_name="c", subcore_axis_name="s")
    f = pl.kernel(
        _route_body,
        out_type=[
            jax.ShapeDtypeStruct((P,), jnp.int32),
            jax.ShapeDtypeStruct((E, NA), jnp.int32),
            jax.ShapeDtypeStruct((NBE,), jnp.int32),
        ],
        mesh=mesh,
        compiler_params=pltpu.CompilerParams(needs_layout_passes=False),
        scratch_types=[
            pltpu.VMEM((NA,), jnp.int32),
            pltpu.VMEM((T,), jnp.int32),
            pltpu.VMEM((NA,), jnp.int32),
            pltpu.VMEM((NBE,), jnp.int32),
            pltpu.VMEM((BLK,), jnp.int32),
        ],
    )
    return f(e1, e2)


NBUF = 3
NB1 = 12
S1 = NB1 * BLK
S2 = P - S1


def _make_gather_body(r0, gw):
    def _gather_body(x_hbm, tok_hbm, xs_hbm, idx_v, rows, gsem, wsem):
        c = lax.axis_index("c")
        s = lax.axis_index("s")
        wid = c * 16 + s
        base = wid * gw
        pltpu.sync_copy(tok_hbm.at[pl.ds(r0 + base, gw)], idx_v)
        nch = gw // GC
        gcp = [None] * NBUF
        wcp = [None] * NBUF
        for j in range(min(NBUF - 1, nch)):
            gcp[j] = pltpu.async_copy(x_hbm.at[idx_v.at[pl.ds(j * GC, GC)]],
                                      rows.at[j], gsem[j])
        for j in range(nch):
            p = j % NBUF
            if j + NBUF - 1 < nch:
                q = (j + NBUF - 1) % NBUF
                if wcp[q] is not None:
                    wcp[q].wait()
                nxt = idx_v.at[pl.ds((j + NBUF - 1) * GC, GC)]
                gcp[q] = pltpu.async_copy(x_hbm.at[nxt], rows.at[q], gsem[q])
            gcp[p].wait()
            wcp[p] = pltpu.async_copy(rows.at[p],
                                      xs_hbm.at[pl.ds(base + j * GC, GC)],
                                      wsem[p])
        for j in range(max(0, nch - NBUF), nch):
            wcp[j % NBUF].wait()
    return _gather_body


def _gather_part(x, tok, r0, ph):
    gw = ph // NW
    mesh = plsc.VectorSubcoreMesh(core_axis_name="c", subcore_axis_name="s")
    f = pl.kernel(
        _make_gather_body(r0, gw),
        out_type=jax.ShapeDtypeStruct((ph, D), jnp.float32),
        mesh=mesh,
        compiler_params=pltpu.CompilerParams(needs_layout_passes=False),
        scratch_types=[
            pltpu.VMEM((gw,), jnp.int32),
            pltpu.VMEM((NBUF, GC, D), jnp.float32),
            [pltpu.SemaphoreType.DMA] * NBUF,
            [pltpu.SemaphoreType.DMA] * NBUF,
        ],
        name=f"gather_part{r0}",
    )
    return f(x, tok)


def _ffn_compute(xs, wg, wu, wd):
    g = lax.dot_general(xs, wg, (((1,), (1,)), ((), ())),
                        preferred_element_type=jnp.float32)
    u = lax.dot_general(xs, wu, (((1,), (1,)), ((), ())),
                        preferred_element_type=jnp.float32)
    h = (g / (1.0 + jnp.exp(-g))) * u
    return lax.dot_general(h, wd, (((1,), (1,)), ((), ())),
                           preferred_element_type=jnp.float32)


def _make_ffn_body_a(boff):
    def body(be_ref, xs_ref, wg_ref, wu_ref, wd_ref, ys_ref):
        @pl.when(pl.program_id(0) + boff < be_ref[NB])
        def _():
            ys_ref[...] = _ffn_compute(xs_ref[...], wg_ref[0], wu_ref[0],
                                       wd_ref[0])
    return body


def _make_ffn_body_b(boff):
    def body(be_ref, xs_ref, wg_ref, wu_ref, wd_ref, prev_ref, ys_ref):
        @pl.when(pl.program_id(0) + boff < be_ref[NB])
        def _():
            ys_ref[...] = _ffn_compute(xs_ref[...], wg_ref[0], wu_ref[0],
                                       wd_ref[0])
    return body


def _ffn_part(be, xs, gate_proj_w, up_proj_w, down_proj_w, boff, nbp,
              ys_prev):
    in_specs = [
        pl.BlockSpec((BLK, D), lambda b, be_ref: (b, 0)),
        pl.BlockSpec((1, DFF, D), lambda b, be_ref: (be_ref[b + boff], 0, 0)),
        pl.BlockSpec((1, DFF, D), lambda b, be_ref: (be_ref[b + boff], 0, 0)),
        pl.BlockSpec((1, D, DFF), lambda b, be_ref: (be_ref[b + boff], 0, 0)),
    ]
    args = [be, xs, gate_proj_w, up_proj_w, down_proj_w]
    aliases = {}
    body = _make_ffn_body_a(boff)
    if ys_prev is not None:
        in_specs.append(pl.BlockSpec(memory_space=pl.ANY))
        args.append(ys_prev)
        aliases = {5: 0}
        body = _make_ffn_body_b(boff)
    grid_spec = pltpu.PrefetchScalarGridSpec(
        num_scalar_prefetch=1,
        grid=(nbp,),
        in_specs=in_specs,
        out_specs=pl.BlockSpec((BLK, D), lambda b, be_ref: (b + boff, 0)),
    )
    return pl.pallas_call(
        body,
        grid_spec=grid_spec,
        out_shape=jax.ShapeDtypeStruct((P, D), jnp.float32),
        input_output_aliases=aliases,
        compiler_params=pltpu.CompilerParams(
            dimension_semantics=("arbitrary",),
        ),
    )(*args)


def _combine_body(ys_hbm, pos_hbm, w1_hbm, w2_hbm, out_hbm,
                  p1_v, p2_v, pv_t, w1_v, w2_v, rows_a, rows_b,
                  sem_a, sem_b, wsem):
    c = lax.axis_index("c")
    s = lax.axis_index("s")
    wid = c * 16 + s
    base = wid * TOK_W
    zeros16 = jnp.zeros((16,), jnp.int32)
    for k in range(TOK_W // 16):
        p1_v[pl.ds(k * 16, 16)] = zeros16
        p2_v[pl.ds(k * 16, 16)] = zeros16
    for w in range(E):
        pltpu.sync_copy(pos_hbm.at[w, pl.ds(base, TOK_W)], pv_t)
        for k in range(TOK_W // 16):
            p1_v[pl.ds(k * 16, 16)] += pv_t[pl.ds(k * 16, 16)]
        pltpu.sync_copy(pos_hbm.at[w, pl.ds(T + base, TOK_W)], pv_t)
        for k in range(TOK_W // 16):
            p2_v[pl.ds(k * 16, 16)] += pv_t[pl.ds(k * 16, 16)]
    pltpu.sync_copy(w1_hbm.at[pl.ds(base, TOK_W)], w1_v)
    pltpu.sync_copy(w2_hbm.at[pl.ds(base, TOK_W)], w2_v)
    wv1 = [w1_v[pl.ds(k * 16, 16)] for k in range(TOK_W // 16)]
    wv2 = [w2_v[pl.ds(k * 16, 16)] for k in range(TOK_W // 16)]

    nch = TOK_W // CC
    ga = [None, None]
    gb = [None, None]
    wb = [None, None]
    ga[0] = pltpu.async_copy(ys_hbm.at[p1_v.at[pl.ds(0, CC)]],
                             rows_a.at[0], sem_a[0])
    gb[0] = pltpu.async_copy(ys_hbm.at[p2_v.at[pl.ds(0, CC)]],
                             rows_b.at[0], sem_b[0])
    for j in range(nch):
        p = j % 2
        q = 1 - p
        if j + 1 < nch:
            if wb[q] is not None:
                wb[q].wait()
            ga[q] = pltpu.async_copy(
                ys_hbm.at[p1_v.at[pl.ds((j + 1) * CC, CC)]],
                rows_a.at[q], sem_a[q])
            gb[q] = pltpu.async_copy(
                ys_hbm.at[p2_v.at[pl.ds((j + 1) * CC, CC)]],
                rows_b.at[q], sem_b[q])
        ga[p].wait()
        gb[p].wait()
        for i in range(CC):
            t_i = j * CC + i
            w1s = wv1[t_i // 16][t_i % 16]
            w2s = wv2[t_i // 16][t_i % 16]

            def inner(m, _):
                for u in range(8):
                    a = rows_a[p, i, pl.ds(m * 128 + u * 16, 16)]
                    b = rows_b[p, i, pl.ds(m * 128 + u * 16, 16)]
                    rows_a[p, i, pl.ds(m * 128 + u * 16, 16)] = (
                        a * w1s + b * w2s)
                return 0
            lax.fori_loop(0, D // 128, inner, 0)
        wb[p] = pltpu.async_copy(rows_a.at[p],
                                 out_hbm.at[pl.ds(base + j * CC, CC)],
                                 wsem[p])
    wb[0].wait()
    wb[1].wait()


def _combine(ys, pos, w1, w2):
    mesh = plsc.VectorSubcoreMesh(core_axis_name="c", subcore_axis_name="s")
    f = pl.kernel(
        _combine_body,
        out_type=jax.ShapeDtypeStruct((T, D), jnp.float32),
        mesh=mesh,
        compiler_params=pltpu.CompilerParams(needs_layout_passes=False),
        scratch_types=[
            pltpu.VMEM((TOK_W,), jnp.int32),
            pltpu.VMEM((TOK_W,), jnp.int32),
            pltpu.VMEM((TOK_W,), jnp.int32),
            pltpu.VMEM((TOK_W,), jnp.float32),
            pltpu.VMEM((TOK_W,), jnp.float32),
            pltpu.VMEM((2, CC, D), jnp.float32),
            pltpu.VMEM((2, CC, D), jnp.float32),
            [pltpu.SemaphoreType.DMA, pltpu.SemaphoreType.DMA],
            [pltpu.SemaphoreType.DMA, pltpu.SemaphoreType.DMA],
            [pltpu.SemaphoreType.DMA, pltpu.SemaphoreType.DMA],
        ],
    )
    return f(ys, pos, w1, w2)


def _route_emul(e1, e2, w1, w2):
    eid = jnp.concatenate([e1, e2])
    wv = jnp.concatenate([w1, w2])
    cnt = jnp.zeros((E,), jnp.int32).at[eid].add(1)
    pcnt = ((cnt + BLK - 1) // BLK) * BLK
    cum = jnp.cumsum(pcnt)
    starts = cum - pcnt
    oh = (eid[:, None] == jnp.arange(E)[None, :]).astype(jnp.int32)
    rank = jnp.take_along_axis(jnp.cumsum(oh, axis=0) - 1, eid[:, None],
                               axis=1)[:, 0]
    pos = starts[eid] + rank
    a = jnp.arange(NA, dtype=jnp.int32)
    tok = jnp.zeros((P,), jnp.int32).at[pos].set(a & (T - 1))
    wsrt = jnp.zeros((P,), jnp.float32).at[pos].set(wv)
    ends = cum // BLK
    b = jnp.arange(NBE, dtype=jnp.int32)
    be = jnp.minimum(jnp.sum(b[:, None] >= ends[None, :], axis=1), E - 1)
    return tok, wsrt, pos.reshape(NA // 16, 16), be.astype(jnp.int32)


def _gather_emul(x, tok):
    return x[tok]


def _combine_emul(ys, pos, w1, w2):
    return w1[:, None] * ys[pos[:T]] + w2[:, None] * ys[pos[T:]]


def kernel(hidden_states, gate_w, gate_proj_w, up_proj_w, down_proj_w):
    b, s, d = hidden_states.shape
    x = hidden_states.reshape(-1, d)
    e1, e2, w1, w2 = _router(x, gate_w)
    tok, pos_part, be = _route(e1, e2)
    xsa = _gather_part(x, tok, 0, S1)
    xsb = _gather_part(x, tok, S1, S2)
    ysa = _ffn_part(be, xsa, gate_proj_w, up_proj_w, down_proj_w, 0, NB1,
                    None)
    ys = _ffn_part(be, xsb, gate_proj_w, up_proj_w, down_proj_w, NB1,
                   NB - NB1, ysa)
    out = _combine(ys, pos_part, w1, w2)
    return out.reshape(b, s, d)
